# baseline (speedup 1.0000x reference)
import functools

import jax
import jax.numpy as jnp
from jax import lax
from jax.experimental import pallas as pl
from jax.experimental.pallas import tpu as pltpu
from jax.experimental.pallas import tpu_sc as plsc

N_USER = 25000
N_ITEM = 25000
N = N_USER + N_ITEM
EMB = 64
H = EMB // 2
E = 800000
TILES = 16
CHUNK = 256
NCH = 196
PAIRS = NCH // 2
IR = CHUNK // 128
EP = TILES * NCH * CHUNK
ET = NCH * CHUNK
IDXROWS_PER_TILE = ET // 128
STRIPE = 3128
STRIPE_LAST = N - 15 * STRIPE
B = 1024
GB = 3 * B // 32

_f32 = jnp.float32
_i32 = jnp.int32


def _spmm_body(ego_a, ego_b, row2d, col2d, valp, zeros, out_a, out_b,
               accum, rowidx0, rowidx1, colidx0, colidx1, rows0, rows1,
               rowsbf0, rowsbf1, vals0, vals1, gsem0, gsem1, ssem0, ssem1):
    c = lax.axis_index("c")
    s = lax.axis_index("s")
    rowidx = [rowidx0, rowidx1]
    colidx = [colidx0, colidx1]
    rows = [rows0, rows1]
    rowsbf = [rowsbf0, rowsbf1]
    vals = [vals0, vals1]

    def run(ego_hbm, out_hbm):
        @pl.when(s < 15)
        def _():
            pltpu.sync_copy(zeros, accum.at[pl.ds(s * STRIPE, STRIPE)])

        @pl.when(s == 15)
        def _():
            pltpu.sync_copy(zeros.at[pl.ds(0, STRIPE_LAST)],
                            accum.at[pl.ds(15 * STRIPE, STRIPE_LAST)])

        plsc.subcore_barrier()

        def load_idx(ci, b):
            sr = s * IDXROWS_PER_TILE + ci * IR
            pltpu.sync_copy(row2d.at[pl.ds(sr, IR)], rowidx[b])
            pltpu.sync_copy(col2d.at[pl.ds(sr, IR)], colidx[b])
            pltpu.sync_copy(valp.at[pl.ds((s * NCH + ci) * CHUNK, CHUNK)],
                            vals[b])

        def fire_gathers(b, sem):
            return [pltpu.async_copy(ego_hbm.at[colidx[b].at[j]],
                                     rowsbf[b].at[pl.ds(j * 128, 128)], sem)
                    for j in range(IR)]

        def drain_gathers(b, sem):
            for j in range(IR):
                pltpu.make_async_copy(
                    ego_hbm.at[colidx[b].at[j]],
                    rowsbf[b].at[pl.ds(j * 128, 128)], sem).wait()

        def fire_scatters(b, sem):
            return [pltpu.async_copy(rows[b].at[pl.ds(j * 128, 128)],
                                     accum.at[rowidx[b].at[j]], sem,
                                     add=True)
                    for j in range(IR)]

        def multiply(b):
            himask = jnp.full((16,), -65536, _i32)

            def mul(g, cc):
                base = g * 16
                v16 = vals[b][pl.ds(base, 16)]
                for k in range(16):
                    bv = jnp.broadcast_to(v16[k], (16,))
                    e = base + k
                    x = rowsbf[b][e, 0:16]
                    pa = lax.bitcast_convert_type(x & himask, _f32)
                    pb = lax.bitcast_convert_type(x << 16, _f32)
                    rows[b][e, 0:16] = pa * bv
                    rows[b][e, 16:32] = pb * bv
                return cc

            lax.fori_loop(0, CHUNK // 16, mul, 0)

        load_idx(0, 0)
        fire_gathers(0, gsem0)

        def pair(k, carry):
            load_idx(2 * k + 1, 1)
            g1 = fire_gathers(1, gsem1)
            drain_gathers(0, gsem0)
            multiply(0)
            s0 = fire_scatters(0, ssem0)
            for cp in g1:
                cp.wait()
            multiply(1)
            for cp in s0:
                cp.wait()
            s1 = fire_scatters(1, ssem1)

            @pl.when(k < PAIRS - 1)
            def _():
                load_idx(2 * k + 2, 0)
                fire_gathers(0, gsem0)

            for cp in s1:
                cp.wait()
            return carry

        lax.fori_loop(0, PAIRS, pair, 0)
        plsc.subcore_barrier()

        @pl.when(s < 15)
        def _():
            pltpu.sync_copy(accum.at[pl.ds(s * STRIPE, STRIPE)],
                            out_hbm.at[pl.ds(s * STRIPE, STRIPE)])

        @pl.when(s == 15)
        def _():
            pltpu.sync_copy(accum.at[pl.ds(15 * STRIPE, STRIPE_LAST)],
                            out_hbm.at[pl.ds(15 * STRIPE, STRIPE_LAST)])

    @pl.when(c == 0)
    def _():
        run(ego_a, out_a)

    @pl.when(c == 1)
    def _():
        run(ego_b, out_b)


def _make_spmm():
    return pl.kernel(
        _spmm_body,
        out_type=[jax.ShapeDtypeStruct((N, H), _f32),
                  jax.ShapeDtypeStruct((N, H), _f32)],
        mesh=plsc.VectorSubcoreMesh(core_axis_name="c", subcore_axis_name="s"),
        scratch_types=[
            pltpu.VMEM_SHARED((N, H), _f32),
            pltpu.VMEM((IR, 128), _i32),
            pltpu.VMEM((IR, 128), _i32),
            pltpu.VMEM((IR, 128), _i32),
            pltpu.VMEM((IR, 128), _i32),
            pltpu.VMEM((CHUNK, H), _f32),
            pltpu.VMEM((CHUNK, H), _f32),
            pltpu.VMEM((CHUNK, 16), _i32),
            pltpu.VMEM((CHUNK, 16), _i32),
            pltpu.VMEM((CHUNK,), _f32),
            pltpu.VMEM((CHUNK,), _f32),
            pltpu.SemaphoreType.DMA,
            pltpu.SemaphoreType.DMA,
            pltpu.SemaphoreType.DMA,
            pltpu.SemaphoreType.DMA,
        ],
        compiler_params=pltpu.CompilerParams(use_tc_tiling_on_sc=False),
    )


_R = 2000


def _pack_cols(x):
    hi = lax.bitcast_convert_type(x[:, :16].astype(jnp.bfloat16),
                                  jnp.uint16).astype(jnp.uint32) << 16
    lo = lax.bitcast_convert_type(x[:, 16:].astype(jnp.bfloat16),
                                  jnp.uint16).astype(jnp.uint32)
    return lax.bitcast_convert_type(hi | lo, jnp.int32)


def _dense_body(sa, sb, ea, eb, wgc, bgc, wbi, bbi, oa, ob, na, nb,
                oabf, obbf):
    side = jnp.concatenate([sa[...], sb[...]], axis=1)
    ego = jnp.concatenate([ea[...], eb[...]], axis=1)
    sum_emb = jnp.dot(side, wgc[...], preferred_element_type=_f32) + bgc[...]
    bi = jnp.dot(ego * side, wbi[...], preferred_element_type=_f32) + bbi[...]
    act = sum_emb + bi
    act = jnp.where(act >= 0, act, 0.2 * act)
    ss = jnp.sum(act * act, axis=1, keepdims=True)
    nrm = act * lax.rsqrt(jnp.maximum(ss, 1e-24))
    oa[...] = act[:, :H]
    ob[...] = act[:, H:]
    na[...] = nrm[:, :H]
    nb[...] = nrm[:, H:]
    oabf[...] = _pack_cols(act[:, :H])
    obbf[...] = _pack_cols(act[:, H:])


def _make_dense():
    blk = pl.BlockSpec((_R, H), lambda i: (i, 0))
    pblk = pl.BlockSpec((_R, 16), lambda i: (i, 0))
    wblk = pl.BlockSpec((EMB, EMB), lambda i: (0, 0))
    bblk = pl.BlockSpec((1, EMB), lambda i: (0, 0))
    return pl.pallas_call(
        _dense_body,
        grid=(N // _R,),
        in_specs=[blk, blk, blk, blk, wblk, bblk, wblk, bblk],
        out_specs=[blk, blk, blk, blk, pblk, pblk],
        out_shape=[jax.ShapeDtypeStruct((N, H), _f32)] * 4
        + [jax.ShapeDtypeStruct((N, 16), _i32)] * 2,
    )


def _final_body(t0, t1, t2, t3, t4, t5, t6, t7, idx3d,
                o0, o1, o2, o3, o4, o5, o6, o7, idx_v, rows_v, sem):
    c = lax.axis_index("c")
    s = lax.axis_index("s")
    wid = s * 2 + c
    pltpu.sync_copy(idx3d.at[wid], idx_v)
    tables = [t0, t1, t2, t3, t4, t5, t6, t7]
    outs = [o0, o1, o2, o3, o4, o5, o6, o7]
    for t in range(8):
        pltpu.async_copy(tables[t].at[idx_v.at[0]], rows_v, sem).wait()
        pltpu.sync_copy(rows_v, outs[t].at[pl.ds(wid * GB, GB)])


def _make_final():
    return pl.kernel(
        _final_body,
        out_type=[jax.ShapeDtypeStruct((3 * B, H), _f32)] * 8,
        mesh=plsc.VectorSubcoreMesh(core_axis_name="c", subcore_axis_name="s"),
        scratch_types=[
            pltpu.VMEM((1, GB), _i32),
            pltpu.VMEM((GB, H), _f32),
            pltpu.SemaphoreType.DMA,
        ],
        compiler_params=pltpu.CompilerParams(use_tc_tiling_on_sc=False),
    )


def kernel(users, pos_items, neg_items, adj_row, adj_col, adj_val,
           user_emb, item_emb,
           W_gc_0, b_gc_0, W_bi_0, b_bi_0,
           W_gc_1, b_gc_1, W_bi_1, b_bi_1,
           W_gc_2, b_gc_2, W_bi_2, b_bi_2):
    ego_a = jnp.concatenate([user_emb[:, :H], item_emb[:, :H]], axis=0)
    ego_b = jnp.concatenate([user_emb[:, H:], item_emb[:, H:]], axis=0)

    pad = EP - E
    row2d = jnp.pad(adj_row.astype(_i32), (0, pad)).reshape(EP // 128, 128)
    col2d = jnp.pad(adj_col.astype(_i32), (0, pad)).reshape(EP // 128, 128)
    valp = jnp.pad(adj_val, (0, pad))
    zeros = jnp.zeros((STRIPE, H), _f32)

    spmm = _make_spmm()
    dense = _make_dense()

    weights = [(W_gc_0, b_gc_0, W_bi_0, b_bi_0),
               (W_gc_1, b_gc_1, W_bi_1, b_bi_1),
               (W_gc_2, b_gc_2, W_bi_2, b_bi_2)]

    tables = [ego_a, ego_b]
    ea, eb = ego_a, ego_b
    ea_bf = _pack_cols(ego_a)
    eb_bf = _pack_cols(ego_b)
    for (wgc, bgc, wbi, bbi) in weights:
        sa, sb = spmm(ea_bf, eb_bf, row2d, col2d, valp, zeros)
        ea, eb, na, nb, ea_bf, eb_bf = dense(sa, sb, ea, eb,
                                             wgc, bgc, wbi, bbi)
        tables += [na, nb]

    idx = jnp.concatenate([users.astype(_i32),
                           pos_items.astype(_i32) + N_USER,
                           neg_items.astype(_i32) + N_USER])
    idx3d = idx.reshape(32, 1, GB)
    outs = _make_final()(*tables, idx3d)
    out = jnp.concatenate(outs, axis=1)
    return out[:B], out[B:2 * B], out[2 * B:]


# speedup vs baseline: 1.2493x; 1.2493x over previous
import functools

import jax
import jax.numpy as jnp
from jax import lax
from jax.experimental import pallas as pl
from jax.experimental.pallas import tpu as pltpu
from jax.experimental.pallas import tpu_sc as plsc

N_USER = 25000
N_ITEM = 25000
N = N_USER + N_ITEM
EMB = 64
H = EMB // 2
E = 800000
TILES = 16
CHUNK = 256
NCH = 196
PAIRS = NCH // 2
IR = CHUNK // 128
EP = TILES * NCH * CHUNK
ET = NCH * CHUNK
IDXROWS_PER_TILE = ET // 128
STRIPE = 3128
STRIPE_LAST = N - 15 * STRIPE
B = 1024
GB = 3 * B // 32

_f32 = jnp.float32
_i32 = jnp.int32


def _spmm_body(ego_a, ego_b, row2d, col2d, valp, zeros, out_a, out_b,
               accum, rowidx0, rowidx1, colidx0, colidx1, rows0, rows1,
               vals0, vals1, gsem0, gsem1, ssem0, ssem1):
    c = lax.axis_index("c")
    s = lax.axis_index("s")
    rowidx = [rowidx0, rowidx1]
    colidx = [colidx0, colidx1]
    rows = [rows0, rows1]
    vals = [vals0, vals1]
    gsem = [gsem0, gsem1]
    ssem = [ssem0, ssem1]

    def run(ego_hbm, out_hbm):
        @pl.when(s < 15)
        def _():
            pltpu.sync_copy(zeros, accum.at[pl.ds(s * STRIPE, STRIPE)])

        @pl.when(s == 15)
        def _():
            pltpu.sync_copy(zeros.at[pl.ds(0, STRIPE_LAST)],
                            accum.at[pl.ds(15 * STRIPE, STRIPE_LAST)])

        plsc.subcore_barrier()

        def load_idx(ci, b):
            sr = s * IDXROWS_PER_TILE + ci * IR
            pltpu.sync_copy(row2d.at[pl.ds(sr, IR)], rowidx[b])
            pltpu.sync_copy(col2d.at[pl.ds(sr, IR)], colidx[b])
            pltpu.sync_copy(valp.at[pl.ds((s * NCH + ci) * CHUNK, CHUNK)],
                            vals[b])

        def fire_gathers(b):
            for j in range(IR):
                pltpu.async_copy(ego_hbm.at[colidx[b].at[j]],
                                 rows[b].at[pl.ds(j * 128, 128)], gsem[b])

        def drain_gathers(b):
            for j in range(IR):
                pltpu.make_async_copy(
                    ego_hbm.at[colidx[b].at[j]],
                    rows[b].at[pl.ds(j * 128, 128)], gsem[b]).wait()

        def fire_scatters(b):
            for j in range(IR):
                pltpu.async_copy(rows[b].at[pl.ds(j * 128, 128)],
                                 accum.at[rowidx[b].at[j]], ssem[b],
                                 add=True)

        def drain_scatters(b):
            for j in range(IR):
                pltpu.make_async_copy(rows[b].at[pl.ds(j * 128, 128)],
                                      accum.at[rowidx[b].at[j]],
                                      ssem[b]).wait()

        def multiply(b):
            def mul(g, cc):
                base = g * 16
                v16 = vals[b][pl.ds(base, 16)]
                for k in range(16):
                    bv = jnp.broadcast_to(v16[k], (16,))
                    e = base + k
                    rows[b][e, 0:16] = rows[b][e, 0:16] * bv
                    rows[b][e, 16:32] = rows[b][e, 16:32] * bv
                return cc

            lax.fori_loop(0, CHUNK // 16, mul, 0)

        load_idx(0, 0)
        fire_gathers(0)

        def step(ci, b, nb, first):
            if not first:
                drain_scatters(nb)

            @pl.when(ci < NCH - 1)
            def _():
                load_idx(ci + 1, nb)
                fire_gathers(nb)

            drain_gathers(b)
            multiply(b)
            fire_scatters(b)

        def pair(k, carry):
            @pl.when(k > 0)
            def _():
                step(2 * k, 0, 1, False)

            @pl.when(k == 0)
            def _():
                step(0, 0, 1, True)

            step(2 * k + 1, 1, 0, False)
            return carry

        lax.fori_loop(0, PAIRS, pair, 0)
        drain_scatters(1)
        plsc.subcore_barrier()

        @pl.when(s < 15)
        def _():
            pltpu.sync_copy(accum.at[pl.ds(s * STRIPE, STRIPE)],
                            out_hbm.at[pl.ds(s * STRIPE, STRIPE)])

        @pl.when(s == 15)
        def _():
            pltpu.sync_copy(accum.at[pl.ds(15 * STRIPE, STRIPE_LAST)],
                            out_hbm.at[pl.ds(15 * STRIPE, STRIPE_LAST)])

    @pl.when(c == 0)
    def _():
        run(ego_a, out_a)

    @pl.when(c == 1)
    def _():
        run(ego_b, out_b)


def _make_spmm():
    return pl.kernel(
        _spmm_body,
        out_type=[jax.ShapeDtypeStruct((N, H), _f32),
                  jax.ShapeDtypeStruct((N, H), _f32)],
        mesh=plsc.VectorSubcoreMesh(core_axis_name="c", subcore_axis_name="s"),
        scratch_types=[
            pltpu.VMEM_SHARED((N, H), _f32),
            pltpu.VMEM((IR, 128), _i32),
            pltpu.VMEM((IR, 128), _i32),
            pltpu.VMEM((IR, 128), _i32),
            pltpu.VMEM((IR, 128), _i32),
            pltpu.VMEM((CHUNK, H), _f32),
            pltpu.VMEM((CHUNK, H), _f32),
            pltpu.VMEM((CHUNK,), _f32),
            pltpu.VMEM((CHUNK,), _f32),
            pltpu.SemaphoreType.DMA,
            pltpu.SemaphoreType.DMA,
            pltpu.SemaphoreType.DMA,
            pltpu.SemaphoreType.DMA,
        ],
        compiler_params=pltpu.CompilerParams(use_tc_tiling_on_sc=False),
    )


_R = 2000


def _pack_cols(x):
    hi = lax.bitcast_convert_type(x[:, :16].astype(jnp.bfloat16),
                                  jnp.uint16).astype(jnp.uint32) << 16
    lo = lax.bitcast_convert_type(x[:, 16:].astype(jnp.bfloat16),
                                  jnp.uint16).astype(jnp.uint32)
    return lax.bitcast_convert_type(hi | lo, jnp.int32)


def _dense_body(sa, sb, ea, eb, wgc, bgc, wbi, bbi, oa, ob, na, nb):
    side = jnp.concatenate([sa[...], sb[...]], axis=1)
    ego = jnp.concatenate([ea[...], eb[...]], axis=1)
    sum_emb = jnp.dot(side, wgc[...], preferred_element_type=_f32) + bgc[...]
    bi = jnp.dot(ego * side, wbi[...], preferred_element_type=_f32) + bbi[...]
    act = sum_emb + bi
    act = jnp.where(act >= 0, act, 0.2 * act)
    ss = jnp.sum(act * act, axis=1, keepdims=True)
    nrm = act * lax.rsqrt(jnp.maximum(ss, 1e-24))
    oa[...] = act[:, :H]
    ob[...] = act[:, H:]
    na[...] = nrm[:, :H]
    nb[...] = nrm[:, H:]


def _make_dense():
    blk = pl.BlockSpec((_R, H), lambda i: (i, 0))
    wblk = pl.BlockSpec((EMB, EMB), lambda i: (0, 0))
    bblk = pl.BlockSpec((1, EMB), lambda i: (0, 0))
    return pl.pallas_call(
        _dense_body,
        grid=(N // _R,),
        in_specs=[blk, blk, blk, blk, wblk, bblk, wblk, bblk],
        out_specs=[blk, blk, blk, blk],
        out_shape=[jax.ShapeDtypeStruct((N, H), _f32)] * 4,
    )


def _final_body(t0, t1, t2, t3, t4, t5, t6, t7, idx3d,
                o0, o1, o2, o3, o4, o5, o6, o7, idx_v, rows_v, sem):
    c = lax.axis_index("c")
    s = lax.axis_index("s")
    wid = s * 2 + c
    pltpu.sync_copy(idx3d.at[wid], idx_v)
    tables = [t0, t1, t2, t3, t4, t5, t6, t7]
    outs = [o0, o1, o2, o3, o4, o5, o6, o7]
    for t in range(8):
        pltpu.async_copy(tables[t].at[idx_v.at[0]], rows_v, sem).wait()
        pltpu.sync_copy(rows_v, outs[t].at[pl.ds(wid * GB, GB)])


def _make_final():
    return pl.kernel(
        _final_body,
        out_type=[jax.ShapeDtypeStruct((3 * B, H), _f32)] * 8,
        mesh=plsc.VectorSubcoreMesh(core_axis_name="c", subcore_axis_name="s"),
        scratch_types=[
            pltpu.VMEM((1, GB), _i32),
            pltpu.VMEM((GB, H), _f32),
            pltpu.SemaphoreType.DMA,
        ],
        compiler_params=pltpu.CompilerParams(use_tc_tiling_on_sc=False),
    )


def kernel(users, pos_items, neg_items, adj_row, adj_col, adj_val,
           user_emb, item_emb,
           W_gc_0, b_gc_0, W_bi_0, b_bi_0,
           W_gc_1, b_gc_1, W_bi_1, b_bi_1,
           W_gc_2, b_gc_2, W_bi_2, b_bi_2):
    ego_a = jnp.concatenate([user_emb[:, :H], item_emb[:, :H]], axis=0)
    ego_b = jnp.concatenate([user_emb[:, H:], item_emb[:, H:]], axis=0)

    pad = EP - E
    row2d = jnp.pad(adj_row.astype(_i32), (0, pad)).reshape(EP // 128, 128)
    col2d = jnp.pad(adj_col.astype(_i32), (0, pad)).reshape(EP // 128, 128)
    valp = jnp.pad(adj_val, (0, pad))
    zeros = jnp.zeros((STRIPE, H), _f32)

    spmm = _make_spmm()
    dense = _make_dense()

    weights = [(W_gc_0, b_gc_0, W_bi_0, b_bi_0),
               (W_gc_1, b_gc_1, W_bi_1, b_bi_1),
               (W_gc_2, b_gc_2, W_bi_2, b_bi_2)]

    tables = [ego_a, ego_b]
    ea, eb = ego_a, ego_b
    for (wgc, bgc, wbi, bbi) in weights:
        sa, sb = spmm(ea, eb, row2d, col2d, valp, zeros)
        ea, eb, na, nb = dense(sa, sb, ea, eb, wgc, bgc, wbi, bbi)
        tables += [na, nb]

    idx = jnp.concatenate([users.astype(_i32),
                           pos_items.astype(_i32) + N_USER,
                           neg_items.astype(_i32) + N_USER])
    idx3d = idx.reshape(32, 1, GB)
    outs = _make_final()(*tables, idx3d)
    out = jnp.concatenate(outs, axis=1)
    return out[:B], out[B:2 * B], out[2 * B:]


# speedup vs baseline: 1.2522x; 1.0023x over previous
import functools

import jax
import jax.numpy as jnp
from jax import lax
from jax.experimental import pallas as pl
from jax.experimental.pallas import tpu as pltpu
from jax.experimental.pallas import tpu_sc as plsc

N_USER = 25000
N_ITEM = 25000
N = N_USER + N_ITEM
EMB = 64
H = EMB // 2
E = 800000
TILES = 16
CHUNK = 256
NCH = 196
PAIRS = NCH // 2
IR = CHUNK // 128
EP = TILES * NCH * CHUNK
ET = NCH * CHUNK
IDXROWS_PER_TILE = ET // 128
STRIPE = 3128
STRIPE_LAST = N - 15 * STRIPE
B = 1024
GB = 3 * B // 32

_f32 = jnp.float32
_i32 = jnp.int32


def _spmm_body(ego_a, ego_b, row2d, col2d, valp, zeros, out_a, out_b,
               accum, rowidx0, rowidx1, colidx0, colidx1, rows0, rows1,
               vals0, vals1, gsem0, gsem1, ssem0, ssem1):
    c = lax.axis_index("c")
    s = lax.axis_index("s")
    rowidx = [rowidx0, rowidx1]
    colidx = [colidx0, colidx1]
    rows = [rows0, rows1]
    vals = [vals0, vals1]
    gsem = [gsem0, gsem1]
    ssem = [ssem0, ssem1]

    def run(ego_hbm, out_hbm):
        @pl.when(s < 15)
        def _():
            pltpu.sync_copy(zeros, accum.at[pl.ds(s * STRIPE, STRIPE)])

        @pl.when(s == 15)
        def _():
            pltpu.sync_copy(zeros.at[pl.ds(0, STRIPE_LAST)],
                            accum.at[pl.ds(15 * STRIPE, STRIPE_LAST)])

        plsc.subcore_barrier()

        def load_idx(ci, b):
            off = (s * NCH + ci) * CHUNK
            pltpu.sync_copy(row2d.at[pl.ds(off, CHUNK)], rowidx[b])
            pltpu.sync_copy(col2d.at[pl.ds(off, CHUNK)], colidx[b])
            pltpu.sync_copy(valp.at[pl.ds(off, CHUNK)], vals[b])

        def fire_gathers(b):
            pltpu.async_copy(ego_hbm.at[colidx[b]], rows[b], gsem[b])

        def drain_gathers(b):
            pltpu.make_async_copy(ego_hbm.at[colidx[b]], rows[b],
                                  gsem[b]).wait()

        def fire_scatters(b):
            pltpu.async_copy(rows[b], accum.at[rowidx[b]], ssem[b],
                             add=True)

        def drain_scatters(b):
            pltpu.make_async_copy(rows[b], accum.at[rowidx[b]],
                                  ssem[b]).wait()

        def multiply(b):
            def mul(g, cc):
                base = g * 16
                v16 = vals[b][pl.ds(base, 16)]
                for k in range(16):
                    bv = jnp.broadcast_to(v16[k], (16,))
                    e = base + k
                    rows[b][e, 0:16] = rows[b][e, 0:16] * bv
                    rows[b][e, 16:32] = rows[b][e, 16:32] * bv
                return cc

            lax.fori_loop(0, CHUNK // 16, mul, 0)

        load_idx(0, 0)
        fire_gathers(0)

        def step(ci, b, nb, first):
            if not first:
                drain_scatters(nb)

            @pl.when(ci < NCH - 1)
            def _():
                load_idx(ci + 1, nb)
                fire_gathers(nb)

            drain_gathers(b)
            multiply(b)
            fire_scatters(b)

        def pair(k, carry):
            @pl.when(k > 0)
            def _():
                step(2 * k, 0, 1, False)

            @pl.when(k == 0)
            def _():
                step(0, 0, 1, True)

            step(2 * k + 1, 1, 0, False)
            return carry

        lax.fori_loop(0, PAIRS, pair, 0)
        drain_scatters(1)
        plsc.subcore_barrier()

        @pl.when(s < 15)
        def _():
            pltpu.sync_copy(accum.at[pl.ds(s * STRIPE, STRIPE)],
                            out_hbm.at[pl.ds(s * STRIPE, STRIPE)])

        @pl.when(s == 15)
        def _():
            pltpu.sync_copy(accum.at[pl.ds(15 * STRIPE, STRIPE_LAST)],
                            out_hbm.at[pl.ds(15 * STRIPE, STRIPE_LAST)])

    @pl.when(c == 0)
    def _():
        run(ego_a, out_a)

    @pl.when(c == 1)
    def _():
        run(ego_b, out_b)


def _make_spmm():
    return pl.kernel(
        _spmm_body,
        out_type=[jax.ShapeDtypeStruct((N, H), _f32),
                  jax.ShapeDtypeStruct((N, H), _f32)],
        mesh=plsc.VectorSubcoreMesh(core_axis_name="c", subcore_axis_name="s"),
        scratch_types=[
            pltpu.VMEM_SHARED((N, H), _f32),
            pltpu.VMEM((CHUNK,), _i32),
            pltpu.VMEM((CHUNK,), _i32),
            pltpu.VMEM((CHUNK,), _i32),
            pltpu.VMEM((CHUNK,), _i32),
            pltpu.VMEM((CHUNK, H), _f32),
            pltpu.VMEM((CHUNK, H), _f32),
            pltpu.VMEM((CHUNK,), _f32),
            pltpu.VMEM((CHUNK,), _f32),
            pltpu.SemaphoreType.DMA,
            pltpu.SemaphoreType.DMA,
            pltpu.SemaphoreType.DMA,
            pltpu.SemaphoreType.DMA,
        ],
        compiler_params=pltpu.CompilerParams(use_tc_tiling_on_sc=False),
    )


_R = 2000


def _pack_cols(x):
    hi = lax.bitcast_convert_type(x[:, :16].astype(jnp.bfloat16),
                                  jnp.uint16).astype(jnp.uint32) << 16
    lo = lax.bitcast_convert_type(x[:, 16:].astype(jnp.bfloat16),
                                  jnp.uint16).astype(jnp.uint32)
    return lax.bitcast_convert_type(hi | lo, jnp.int32)


def _dense_body(sa, sb, ea, eb, wgc, bgc, wbi, bbi, oa, ob, na, nb):
    side = jnp.concatenate([sa[...], sb[...]], axis=1)
    ego = jnp.concatenate([ea[...], eb[...]], axis=1)
    sum_emb = jnp.dot(side, wgc[...], preferred_element_type=_f32) + bgc[...]
    bi = jnp.dot(ego * side, wbi[...], preferred_element_type=_f32) + bbi[...]
    act = sum_emb + bi
    act = jnp.where(act >= 0, act, 0.2 * act)
    ss = jnp.sum(act * act, axis=1, keepdims=True)
    nrm = act * lax.rsqrt(jnp.maximum(ss, 1e-24))
    oa[...] = act[:, :H]
    ob[...] = act[:, H:]
    na[...] = nrm[:, :H]
    nb[...] = nrm[:, H:]


def _make_dense():
    blk = pl.BlockSpec((_R, H), lambda i: (i, 0))
    wblk = pl.BlockSpec((EMB, EMB), lambda i: (0, 0))
    bblk = pl.BlockSpec((1, EMB), lambda i: (0, 0))
    return pl.pallas_call(
        _dense_body,
        grid=(N // _R,),
        in_specs=[blk, blk, blk, blk, wblk, bblk, wblk, bblk],
        out_specs=[blk, blk, blk, blk],
        out_shape=[jax.ShapeDtypeStruct((N, H), _f32)] * 4,
    )


def _final_body(t0, t1, t2, t3, t4, t5, t6, t7, idx3d,
                o0, o1, o2, o3, o4, o5, o6, o7, idx_v, rows_v, sem):
    c = lax.axis_index("c")
    s = lax.axis_index("s")
    wid = s * 2 + c
    pltpu.sync_copy(idx3d.at[wid], idx_v)
    tables = [t0, t1, t2, t3, t4, t5, t6, t7]
    outs = [o0, o1, o2, o3, o4, o5, o6, o7]
    for t in range(8):
        pltpu.async_copy(tables[t].at[idx_v.at[0]], rows_v, sem).wait()
        pltpu.sync_copy(rows_v, outs[t].at[pl.ds(wid * GB, GB)])


def _make_final():
    return pl.kernel(
        _final_body,
        out_type=[jax.ShapeDtypeStruct((3 * B, H), _f32)] * 8,
        mesh=plsc.VectorSubcoreMesh(core_axis_name="c", subcore_axis_name="s"),
        scratch_types=[
            pltpu.VMEM((1, GB), _i32),
            pltpu.VMEM((GB, H), _f32),
            pltpu.SemaphoreType.DMA,
        ],
        compiler_params=pltpu.CompilerParams(use_tc_tiling_on_sc=False),
    )


def kernel(users, pos_items, neg_items, adj_row, adj_col, adj_val,
           user_emb, item_emb,
           W_gc_0, b_gc_0, W_bi_0, b_bi_0,
           W_gc_1, b_gc_1, W_bi_1, b_bi_1,
           W_gc_2, b_gc_2, W_bi_2, b_bi_2):
    ego_a = jnp.concatenate([user_emb[:, :H], item_emb[:, :H]], axis=0)
    ego_b = jnp.concatenate([user_emb[:, H:], item_emb[:, H:]], axis=0)

    pad = EP - E
    row2d = jnp.pad(adj_row.astype(_i32), (0, pad))
    col2d = jnp.pad(adj_col.astype(_i32), (0, pad))
    valp = jnp.pad(adj_val, (0, pad))
    zeros = jnp.zeros((STRIPE, H), _f32)

    spmm = _make_spmm()
    dense = _make_dense()

    weights = [(W_gc_0, b_gc_0, W_bi_0, b_bi_0),
               (W_gc_1, b_gc_1, W_bi_1, b_bi_1),
               (W_gc_2, b_gc_2, W_bi_2, b_bi_2)]

    tables = [ego_a, ego_b]
    ea, eb = ego_a, ego_b
    for (wgc, bgc, wbi, bbi) in weights:
        sa, sb = spmm(ea, eb, row2d, col2d, valp, zeros)
        ea, eb, na, nb = dense(sa, sb, ea, eb, wgc, bgc, wbi, bbi)
        tables += [na, nb]

    idx = jnp.concatenate([users.astype(_i32),
                           pos_items.astype(_i32) + N_USER,
                           neg_items.astype(_i32) + N_USER])
    idx3d = idx.reshape(32, 1, GB)
    outs = _make_final()(*tables, idx3d)
    out = jnp.concatenate(outs, axis=1)
    return out[:B], out[B:2 * B], out[2 * B:]


# speedup vs baseline: 1.2920x; 1.0318x over previous
import functools

import jax
import jax.numpy as jnp
from jax import lax
from jax.experimental import pallas as pl
from jax.experimental.pallas import tpu as pltpu
from jax.experimental.pallas import tpu_sc as plsc

N_USER = 25000
N_ITEM = 25000
N = N_USER + N_ITEM
EMB = 64
H = EMB // 2
E = 800000
TILES = 16
CHUNK = 256
NCH = 196
PAIRS = NCH // 2
IR = CHUNK // 128
EP = TILES * NCH * CHUNK
ET = NCH * CHUNK
IDXROWS_PER_TILE = ET // 128
STRIPE = 3128
STRIPE_LAST = N - 15 * STRIPE
B = 1024
GB = 3 * B // 32

_f32 = jnp.float32
_i32 = jnp.int32


def _spmm_body(ego_a, ego_b, row2d, col2d, valp, zeros, out_a, out_b,
               accum, rowidx0, rowidx1, colidx0, colidx1, rows0, rows1,
               vals0, vals1, gsem0, gsem1, ssem0, ssem1):
    c = lax.axis_index("c")
    s = lax.axis_index("s")
    rowidx = [rowidx0, rowidx1]
    colidx = [colidx0, colidx1]
    rows = [rows0, rows1]
    vals = [vals0, vals1]
    gsem = [gsem0, gsem1]
    ssem = [ssem0, ssem1]

    def run(ego_hbm, out_hbm):
        @pl.when(s < 15)
        def _():
            pltpu.sync_copy(zeros, accum.at[pl.ds(s * STRIPE, STRIPE)])

        @pl.when(s == 15)
        def _():
            pltpu.sync_copy(zeros.at[pl.ds(0, STRIPE_LAST)],
                            accum.at[pl.ds(15 * STRIPE, STRIPE_LAST)])

        plsc.subcore_barrier()

        def load_idx(ci, b):
            off = (s * NCH + ci) * CHUNK
            pltpu.sync_copy(row2d.at[pl.ds(off, CHUNK)], rowidx[b])
            pltpu.sync_copy(col2d.at[pl.ds(off, CHUNK)], colidx[b])
            pltpu.sync_copy(valp.at[pl.ds(off, CHUNK)], vals[b])

        def fire_gathers(b):
            pltpu.async_copy(ego_hbm.at[colidx[b]], rows[b], gsem[b])

        def drain_gathers(b):
            pltpu.make_async_copy(ego_hbm.at[colidx[b]], rows[b],
                                  gsem[b]).wait()

        def fire_scatters(b):
            pltpu.async_copy(rows[b], accum.at[rowidx[b]], ssem[b],
                             add=True)

        def drain_scatters(b):
            pltpu.make_async_copy(rows[b], accum.at[rowidx[b]],
                                  ssem[b]).wait()

        def multiply(b):
            def mul(g, cc):
                base = g * 16
                v16 = vals[b][pl.ds(base, 16)]
                for k in range(16):
                    bv = jnp.broadcast_to(v16[k], (16,))
                    e = base + k
                    rows[b][e, 0:16] = rows[b][e, 0:16] * bv
                    rows[b][e, 16:32] = rows[b][e, 16:32] * bv
                return cc

            lax.fori_loop(0, CHUNK // 16, mul, 0)

        load_idx(0, 0)
        fire_gathers(0)

        def step(ci, b, nb, first):
            if not first:
                drain_scatters(nb)

            @pl.when(ci < NCH - 1)
            def _():
                load_idx(ci + 1, nb)
                fire_gathers(nb)

            drain_gathers(b)
            multiply(b)
            fire_scatters(b)

        def pair(k, carry):
            @pl.when(k > 0)
            def _():
                step(2 * k, 0, 1, False)

            @pl.when(k == 0)
            def _():
                step(0, 0, 1, True)

            step(2 * k + 1, 1, 0, False)
            return carry

        lax.fori_loop(0, PAIRS, pair, 0)
        drain_scatters(1)
        plsc.subcore_barrier()

        @pl.when(s < 15)
        def _():
            pltpu.sync_copy(accum.at[pl.ds(s * STRIPE, STRIPE)],
                            out_hbm.at[pl.ds(s * STRIPE, STRIPE)])

        @pl.when(s == 15)
        def _():
            pltpu.sync_copy(accum.at[pl.ds(15 * STRIPE, STRIPE_LAST)],
                            out_hbm.at[pl.ds(15 * STRIPE, STRIPE_LAST)])

    @pl.when(c == 0)
    def _():
        run(ego_a, out_a)

    @pl.when(c == 1)
    def _():
        run(ego_b, out_b)


def _make_spmm():
    return pl.kernel(
        _spmm_body,
        out_type=[jax.ShapeDtypeStruct((N, H), _f32),
                  jax.ShapeDtypeStruct((N, H), _f32)],
        mesh=plsc.VectorSubcoreMesh(core_axis_name="c", subcore_axis_name="s"),
        scratch_types=[
            pltpu.VMEM_SHARED((N, H), _f32),
            pltpu.VMEM((CHUNK,), _i32),
            pltpu.VMEM((CHUNK,), _i32),
            pltpu.VMEM((CHUNK,), _i32),
            pltpu.VMEM((CHUNK,), _i32),
            pltpu.VMEM((CHUNK, H), _f32),
            pltpu.VMEM((CHUNK, H), _f32),
            pltpu.VMEM((CHUNK,), _f32),
            pltpu.VMEM((CHUNK,), _f32),
            pltpu.SemaphoreType.DMA,
            pltpu.SemaphoreType.DMA,
            pltpu.SemaphoreType.DMA,
            pltpu.SemaphoreType.DMA,
        ],
        compiler_params=pltpu.CompilerParams(use_tc_tiling_on_sc=False),
    )


_R = 2000


def _pack_cols(x):
    hi = lax.bitcast_convert_type(x[:, :16].astype(jnp.bfloat16),
                                  jnp.uint16).astype(jnp.uint32) << 16
    lo = lax.bitcast_convert_type(x[:, 16:].astype(jnp.bfloat16),
                                  jnp.uint16).astype(jnp.uint32)
    return lax.bitcast_convert_type(hi | lo, jnp.int32)


def _dense_body(sa, sb, ea, eb, wgc, bgc, wbi, bbi, oa, ob):
    side = jnp.concatenate([sa[...], sb[...]], axis=1)
    ego = jnp.concatenate([ea[...], eb[...]], axis=1)
    sum_emb = jnp.dot(side, wgc[...], preferred_element_type=_f32) + bgc[...]
    bi = jnp.dot(ego * side, wbi[...], preferred_element_type=_f32) + bbi[...]
    act = sum_emb + bi
    act = jnp.where(act >= 0, act, 0.2 * act)
    oa[...] = act[:, :H]
    ob[...] = act[:, H:]


def _make_dense():
    blk = pl.BlockSpec((_R, H), lambda i: (i, 0))
    wblk = pl.BlockSpec((EMB, EMB), lambda i: (0, 0))
    bblk = pl.BlockSpec((1, EMB), lambda i: (0, 0))
    return pl.pallas_call(
        _dense_body,
        grid=(N // _R,),
        in_specs=[blk, blk, blk, blk, wblk, bblk, wblk, bblk],
        out_specs=[blk, blk],
        out_shape=[jax.ShapeDtypeStruct((N, H), _f32)] * 2,
    )


def _norm_body(x_ref, o_ref):
    x = x_ref[...]
    o_ref[:, 0:EMB] = x[:, 0:EMB]
    for k in range(1, 4):
        blk = x[:, k * EMB:(k + 1) * EMB]
        ss = jnp.sum(blk * blk, axis=1, keepdims=True)
        o_ref[:, k * EMB:(k + 1) * EMB] = blk * lax.rsqrt(
            jnp.maximum(ss, 1e-24))


def _make_norm():
    blk = pl.BlockSpec((3 * B // 2, 4 * EMB), lambda i: (i, 0))
    return pl.pallas_call(
        _norm_body,
        grid=(2,),
        in_specs=[blk],
        out_specs=blk,
        out_shape=jax.ShapeDtypeStruct((3 * B, 4 * EMB), _f32),
    )


def _final_body(t0, t1, t2, t3, t4, t5, t6, t7, idx3d,
                o0, o1, o2, o3, o4, o5, o6, o7, idx_v, rows_v, sem):
    c = lax.axis_index("c")
    s = lax.axis_index("s")
    wid = s * 2 + c
    pltpu.sync_copy(idx3d.at[wid], idx_v)
    tables = [t0, t1, t2, t3, t4, t5, t6, t7]
    outs = [o0, o1, o2, o3, o4, o5, o6, o7]
    for t in range(8):
        pltpu.async_copy(tables[t].at[idx_v.at[0]], rows_v, sem).wait()
        pltpu.sync_copy(rows_v, outs[t].at[pl.ds(wid * GB, GB)])


def _make_final():
    return pl.kernel(
        _final_body,
        out_type=[jax.ShapeDtypeStruct((3 * B, H), _f32)] * 8,
        mesh=plsc.VectorSubcoreMesh(core_axis_name="c", subcore_axis_name="s"),
        scratch_types=[
            pltpu.VMEM((1, GB), _i32),
            pltpu.VMEM((GB, H), _f32),
            pltpu.SemaphoreType.DMA,
        ],
        compiler_params=pltpu.CompilerParams(use_tc_tiling_on_sc=False),
    )


def kernel(users, pos_items, neg_items, adj_row, adj_col, adj_val,
           user_emb, item_emb,
           W_gc_0, b_gc_0, W_bi_0, b_bi_0,
           W_gc_1, b_gc_1, W_bi_1, b_bi_1,
           W_gc_2, b_gc_2, W_bi_2, b_bi_2):
    ego_a = jnp.concatenate([user_emb[:, :H], item_emb[:, :H]], axis=0)
    ego_b = jnp.concatenate([user_emb[:, H:], item_emb[:, H:]], axis=0)

    pad = EP - E
    row2d = jnp.pad(adj_row.astype(_i32), (0, pad))
    col2d = jnp.pad(adj_col.astype(_i32), (0, pad))
    valp = jnp.pad(adj_val, (0, pad))
    zeros = jnp.zeros((STRIPE, H), _f32)

    spmm = _make_spmm()
    dense = _make_dense()

    weights = [(W_gc_0, b_gc_0, W_bi_0, b_bi_0),
               (W_gc_1, b_gc_1, W_bi_1, b_bi_1),
               (W_gc_2, b_gc_2, W_bi_2, b_bi_2)]

    tables = [ego_a, ego_b]
    ea, eb = ego_a, ego_b
    for (wgc, bgc, wbi, bbi) in weights:
        sa, sb = spmm(ea, eb, row2d, col2d, valp, zeros)
        ea, eb = dense(sa, sb, ea, eb, wgc, bgc, wbi, bbi)
        tables += [ea, eb]

    idx = jnp.concatenate([users.astype(_i32),
                           pos_items.astype(_i32) + N_USER,
                           neg_items.astype(_i32) + N_USER])
    idx3d = idx.reshape(32, 1, GB)
    outs = _make_final()(*tables, idx3d)
    out = _make_norm()(jnp.concatenate(outs, axis=1))
    return out[:B], out[B:2 * B], out[2 * B:]


# speedup vs baseline: 1.2974x; 1.0042x over previous
import functools

import jax
import jax.numpy as jnp
from jax import lax
from jax.experimental import pallas as pl
from jax.experimental.pallas import tpu as pltpu
from jax.experimental.pallas import tpu_sc as plsc

N_USER = 25000
N_ITEM = 25000
N = N_USER + N_ITEM
EMB = 64
H = EMB // 2
E = 800000
TILES = 16
CHUNK = 256
NCH = 196
PAIRS = NCH // 2
IR = CHUNK // 128
EP = TILES * NCH * CHUNK
ET = NCH * CHUNK
IDXROWS_PER_TILE = ET // 128
STRIPE = 3128
STRIPE_LAST = N - 15 * STRIPE
B = 1024
GB = 3 * B // 32

_f32 = jnp.float32
_i32 = jnp.int32


def _spmm_body(ego_a, ego_b, row2d, col2d, valp, zeros, out_a, out_b,
               accum, rowidx0, rowidx1, colidx0, colidx1, rows0, rows1,
               vals0, vals1, gsem0, gsem1, ssem0, ssem1):
    c = lax.axis_index("c")
    s = lax.axis_index("s")
    rowidx = [rowidx0, rowidx1]
    colidx = [colidx0, colidx1]
    rows = [rows0, rows1]
    vals = [vals0, vals1]
    gsem = [gsem0, gsem1]
    ssem = [ssem0, ssem1]

    def run(ego_hbm, out_hbm):
        @pl.when(s < 15)
        def _():
            pltpu.sync_copy(zeros, accum.at[pl.ds(s * STRIPE, STRIPE)])

        @pl.when(s == 15)
        def _():
            pltpu.sync_copy(zeros.at[pl.ds(0, STRIPE_LAST)],
                            accum.at[pl.ds(15 * STRIPE, STRIPE_LAST)])

        plsc.subcore_barrier()

        def load_idx(ci, b):
            off = (s * NCH + ci) * CHUNK
            pltpu.sync_copy(row2d.at[pl.ds(off, CHUNK)], rowidx[b])
            pltpu.sync_copy(col2d.at[pl.ds(off, CHUNK)], colidx[b])
            pltpu.sync_copy(valp.at[pl.ds(off, CHUNK)], vals[b])

        def fire_gathers(b):
            pltpu.async_copy(ego_hbm.at[colidx[b]], rows[b], gsem[b])

        def drain_gathers(b):
            pltpu.make_async_copy(ego_hbm.at[colidx[b]], rows[b],
                                  gsem[b]).wait()

        def fire_scatters(b):
            pltpu.async_copy(rows[b], accum.at[rowidx[b]], ssem[b],
                             add=True)

        def drain_scatters(b):
            pltpu.make_async_copy(rows[b], accum.at[rowidx[b]],
                                  ssem[b]).wait()

        def multiply(b):
            def mul(g, cc):
                base = g * 16
                v16 = vals[b][pl.ds(base, 16)]
                for k in range(16):
                    bv = jnp.broadcast_to(v16[k], (16,))
                    e = base + k
                    rows[b][e, 0:16] = rows[b][e, 0:16] * bv
                    rows[b][e, 16:32] = rows[b][e, 16:32] * bv
                return cc

            lax.fori_loop(0, CHUNK // 16, mul, 0)

        load_idx(0, 0)
        fire_gathers(0)

        def step(ci, b, nb, first):
            if not first:
                drain_scatters(nb)

            @pl.when(ci < NCH - 1)
            def _():
                load_idx(ci + 1, nb)
                fire_gathers(nb)

            drain_gathers(b)
            multiply(b)
            fire_scatters(b)

        def pair(k, carry):
            @pl.when(k > 0)
            def _():
                step(2 * k, 0, 1, False)

            @pl.when(k == 0)
            def _():
                step(0, 0, 1, True)

            step(2 * k + 1, 1, 0, False)
            return carry

        lax.fori_loop(0, PAIRS, pair, 0)
        drain_scatters(1)
        plsc.subcore_barrier()

        @pl.when(s < 15)
        def _():
            pltpu.sync_copy(accum.at[pl.ds(s * STRIPE, STRIPE)],
                            out_hbm.at[pl.ds(s * STRIPE, STRIPE)])

        @pl.when(s == 15)
        def _():
            pltpu.sync_copy(accum.at[pl.ds(15 * STRIPE, STRIPE_LAST)],
                            out_hbm.at[pl.ds(15 * STRIPE, STRIPE_LAST)])

    @pl.when(c == 0)
    def _():
        run(ego_a, out_a)

    @pl.when(c == 1)
    def _():
        run(ego_b, out_b)


def _make_spmm():
    return pl.kernel(
        _spmm_body,
        out_type=[jax.ShapeDtypeStruct((N, H), _f32),
                  jax.ShapeDtypeStruct((N, H), _f32)],
        mesh=plsc.VectorSubcoreMesh(core_axis_name="c", subcore_axis_name="s"),
        scratch_types=[
            pltpu.VMEM_SHARED((N, H), _f32),
            pltpu.VMEM((CHUNK,), _i32),
            pltpu.VMEM((CHUNK,), _i32),
            pltpu.VMEM((CHUNK,), _i32),
            pltpu.VMEM((CHUNK,), _i32),
            pltpu.VMEM((CHUNK, H), _f32),
            pltpu.VMEM((CHUNK, H), _f32),
            pltpu.VMEM((CHUNK,), _f32),
            pltpu.VMEM((CHUNK,), _f32),
            pltpu.SemaphoreType.DMA,
            pltpu.SemaphoreType.DMA,
            pltpu.SemaphoreType.DMA,
            pltpu.SemaphoreType.DMA,
        ],
        compiler_params=pltpu.CompilerParams(use_tc_tiling_on_sc=False),
    )


_R = 5000


def _pack_cols(x):
    hi = lax.bitcast_convert_type(x[:, :16].astype(jnp.bfloat16),
                                  jnp.uint16).astype(jnp.uint32) << 16
    lo = lax.bitcast_convert_type(x[:, 16:].astype(jnp.bfloat16),
                                  jnp.uint16).astype(jnp.uint32)
    return lax.bitcast_convert_type(hi | lo, jnp.int32)


def _dense_body(sa, sb, ea, eb, wgc, bgc, wbi, bbi, oa, ob):
    side = jnp.concatenate([sa[...], sb[...]], axis=1)
    ego = jnp.concatenate([ea[...], eb[...]], axis=1)
    sum_emb = jnp.dot(side, wgc[...], preferred_element_type=_f32) + bgc[...]
    bi = jnp.dot(ego * side, wbi[...], preferred_element_type=_f32) + bbi[...]
    act = sum_emb + bi
    act = jnp.where(act >= 0, act, 0.2 * act)
    oa[...] = act[:, :H]
    ob[...] = act[:, H:]


def _make_dense():
    blk = pl.BlockSpec((_R, H), lambda i: (i, 0))
    wblk = pl.BlockSpec((EMB, EMB), lambda i: (0, 0))
    bblk = pl.BlockSpec((1, EMB), lambda i: (0, 0))
    return pl.pallas_call(
        _dense_body,
        grid=(N // _R,),
        in_specs=[blk, blk, blk, blk, wblk, bblk, wblk, bblk],
        out_specs=[blk, blk],
        out_shape=[jax.ShapeDtypeStruct((N, H), _f32)] * 2,
    )


def _norm_body(x_ref, o_ref):
    x = x_ref[...]
    o_ref[:, 0:EMB] = x[:, 0:EMB]
    for k in range(1, 4):
        blk = x[:, k * EMB:(k + 1) * EMB]
        ss = jnp.sum(blk * blk, axis=1, keepdims=True)
        o_ref[:, k * EMB:(k + 1) * EMB] = blk * lax.rsqrt(
            jnp.maximum(ss, 1e-24))


def _make_norm():
    blk = pl.BlockSpec((3 * B // 2, 4 * EMB), lambda i: (i, 0))
    return pl.pallas_call(
        _norm_body,
        grid=(2,),
        in_specs=[blk],
        out_specs=blk,
        out_shape=jax.ShapeDtypeStruct((3 * B, 4 * EMB), _f32),
    )


def _final_body(t0, t1, t2, t3, t4, t5, t6, t7, idx3d,
                o0, o1, o2, o3, o4, o5, o6, o7, idx_v, rows_v, sem):
    c = lax.axis_index("c")
    s = lax.axis_index("s")
    wid = s * 2 + c
    pltpu.sync_copy(idx3d.at[wid], idx_v)
    tables = [t0, t1, t2, t3, t4, t5, t6, t7]
    outs = [o0, o1, o2, o3, o4, o5, o6, o7]
    for t in range(8):
        pltpu.async_copy(tables[t].at[idx_v.at[0]], rows_v, sem).wait()
        pltpu.sync_copy(rows_v, outs[t].at[pl.ds(wid * GB, GB)])


def _make_final():
    return pl.kernel(
        _final_body,
        out_type=[jax.ShapeDtypeStruct((3 * B, H), _f32)] * 8,
        mesh=plsc.VectorSubcoreMesh(core_axis_name="c", subcore_axis_name="s"),
        scratch_types=[
            pltpu.VMEM((1, GB), _i32),
            pltpu.VMEM((GB, H), _f32),
            pltpu.SemaphoreType.DMA,
        ],
        compiler_params=pltpu.CompilerParams(use_tc_tiling_on_sc=False),
    )


def kernel(users, pos_items, neg_items, adj_row, adj_col, adj_val,
           user_emb, item_emb,
           W_gc_0, b_gc_0, W_bi_0, b_bi_0,
           W_gc_1, b_gc_1, W_bi_1, b_bi_1,
           W_gc_2, b_gc_2, W_bi_2, b_bi_2):
    ego_a = jnp.concatenate([user_emb[:, :H], item_emb[:, :H]], axis=0)
    ego_b = jnp.concatenate([user_emb[:, H:], item_emb[:, H:]], axis=0)

    pad = EP - E
    row2d = jnp.pad(adj_row.astype(_i32), (0, pad))
    col2d = jnp.pad(adj_col.astype(_i32), (0, pad))
    valp = jnp.pad(adj_val, (0, pad))
    zeros = jnp.zeros((STRIPE, H), _f32)

    spmm = _make_spmm()
    dense = _make_dense()

    weights = [(W_gc_0, b_gc_0, W_bi_0, b_bi_0),
               (W_gc_1, b_gc_1, W_bi_1, b_bi_1),
               (W_gc_2, b_gc_2, W_bi_2, b_bi_2)]

    tables = [ego_a, ego_b]
    ea, eb = ego_a, ego_b
    for (wgc, bgc, wbi, bbi) in weights:
        sa, sb = spmm(ea, eb, row2d, col2d, valp, zeros)
        ea, eb = dense(sa, sb, ea, eb, wgc, bgc, wbi, bbi)
        tables += [ea, eb]

    idx = jnp.concatenate([users.astype(_i32),
                           pos_items.astype(_i32) + N_USER,
                           neg_items.astype(_i32) + N_USER])
    idx3d = idx.reshape(32, 1, GB)
    outs = _make_final()(*tables, idx3d)
    out = _make_norm()(jnp.concatenate(outs, axis=1))
    return out[:B], out[B:2 * B], out[2 * B:]
